# baseline (speedup 1.0000x reference)
import jax
import jax.numpy as jnp
import numpy as np
from jax import lax
from jax.experimental import pallas as pl
from jax.experimental.pallas import tpu as pltpu

_HIDDEN = 32
_FRAC = 0.35
_MIN_K = 6
_RB = 64
_CB = 128
_CHUNK = 1024
_NCH = 8
_NS = 4

_INT_MIN = np.int32(-2147483648)

_ROUNDS = [(29, 3), (26, 3), (23, 3), (20, 3), (17, 3), (14, 3), (11, 3),
           (8, 3), (5, 3), (2, 3), (0, 2)]


def _row_kernel(len_ref, x0_ref, x1_ref, x2_ref, x3_ref,
                W1_ref, b1_ref, W2_ref, out_ref, s_scr):
    b = pl.program_id(0)
    T = _RB * _CB
    l = len_ref[b]
    xrefs = [x0_ref, x1_ref, x2_ref, x3_ref]
    per_stream = _NCH // _NS

    for c in range(_NCH):
        @pl.when(l > c * _CHUNK)
        def _():
            half = xrefs[c // per_stream]
            cc_ = c % per_stream
            xc = half[0][cc_ * _CHUNK:(cc_ + 1) * _CHUNK, :]
            h = jnp.tanh(
                jnp.dot(xc, W1_ref[...], preferred_element_type=jnp.float32)
                + b1_ref[...]
            )
            sc = jnp.dot(h, W2_ref[...], preferred_element_type=jnp.float32)
            s_scr[c * 8:(c + 1) * 8, :] = sc.reshape(8, _CB)

    r = lax.broadcasted_iota(jnp.int32, (_RB, _CB), 0)
    cc0 = lax.broadcasted_iota(jnp.int32, (_RB, _CB), 1)
    t = r * _CB + cc0
    s2 = jnp.where(t < l, s_scr[...], -jnp.inf)

    i = lax.bitcast_convert_type(s2, jnp.int32)
    skey = i ^ (lax.shift_right_arithmetic(i, 31) & jnp.int32(0x7FFFFFFF))

    lf = l.astype(jnp.float32) * jnp.float32(_FRAC)
    ki = lf.astype(jnp.int32)
    ki = ki + (ki.astype(jnp.float32) < lf).astype(jnp.int32)
    k = jnp.clip(ki, _MIN_K, T)

    p = jnp.int32(0)
    for shift, width in _ROUNDS:
        n = (1 << width) - 1
        oks = []
        for j in range(1, n + 1):
            cpat = p | (jnp.int32(j) << shift)
            cval = cpat ^ jnp.int32(_INT_MIN)
            cnt = jnp.sum((skey >= cval).astype(jnp.int32))
            oks.append((cnt >= k).astype(jnp.int32))
        j_star = oks[0]
        for o in oks[1:]:
            j_star = j_star + o
        p = p | (j_star << shift)
    theta = p ^ jnp.int32(_INT_MIN)

    cgt = jnp.sum((skey > theta).astype(jnp.int32))
    m = (k - cgt).astype(jnp.float32)

    eq = (skey == theta).astype(jnp.float32)
    cc = lax.broadcasted_iota(jnp.int32, (_CB, _CB), 0)
    cr = lax.broadcasted_iota(jnp.int32, (_CB, _CB), 1)
    lt_incl = (cc <= cr).astype(jnp.float32)
    lane_incl = jnp.dot(eq, lt_incl, preferred_element_type=jnp.float32)
    row_tot = jnp.sum(eq, axis=1, keepdims=True)
    ar = lax.broadcasted_iota(jnp.int32, (_RB, _RB), 0)
    ac = lax.broadcasted_iota(jnp.int32, (_RB, _RB), 1)
    strict = (ac < ar).astype(jnp.float32)
    row_excl = jnp.dot(strict, row_tot, preferred_element_type=jnp.float32)
    rank_excl = row_excl + lane_incl - eq

    w = jnp.where(
        (skey > theta) | ((skey == theta) & (rank_excl < m)),
        jnp.float32(1.0),
        jnp.float32(0.0),
    )

    wrow = w.reshape(1, T)
    Q = T // _NS
    pooled = jnp.dot(wrow[:, 0:Q], x0_ref[0],
                     preferred_element_type=jnp.float32)
    for s, ref in enumerate(xrefs[1:], start=1):
        pooled = pooled + jnp.dot(
            wrow[:, s * Q:(s + 1) * Q], ref[0],
            preferred_element_type=jnp.float32)
    out_ref[0] = pooled / k.astype(jnp.float32)


def kernel(x, lengths, W1, b1, W2):
    B, T, D = x.shape
    lengths = lengths.astype(jnp.int32)
    b1r = b1.reshape(1, _HIDDEN).astype(jnp.float32)
    xq = x.reshape(_NS * B, T // _NS, D)
    qspecs = [
        pl.BlockSpec((1, T // _NS, D),
                     (lambda s: (lambda b: (_NS * b + s, 0, 0)))(s))
        for s in range(_NS)
    ]
    return pl.pallas_call(
        _row_kernel,
        grid=(B,),
        in_specs=[pl.BlockSpec(memory_space=pltpu.SMEM)] + qspecs + [
            pl.BlockSpec((D, _HIDDEN), lambda b: (0, 0)),
            pl.BlockSpec((1, _HIDDEN), lambda b: (0, 0)),
            pl.BlockSpec((_HIDDEN, 1), lambda b: (0, 0)),
        ],
        out_specs=pl.BlockSpec((1, 1, D), lambda b: (b, 0, 0)),
        out_shape=jax.ShapeDtypeStruct((B, 1, D), jnp.float32),
        scratch_shapes=[pltpu.VMEM((_RB, _CB), jnp.float32)],
    )(lengths, xq, xq, xq, xq, W1, b1r, W2).reshape(B, D)


# speedup vs baseline: 1.0025x; 1.0025x over previous
import jax
import jax.numpy as jnp
import numpy as np
from jax import lax
from jax.experimental import pallas as pl
from jax.experimental.pallas import tpu as pltpu

_HIDDEN = 32
_FRAC = 0.35
_MIN_K = 6
_RB = 64
_CB = 128
_CHUNK = 1024
_NCH = 8

_INT_MIN = np.int32(-2147483648)

_ROUNDS = [(29, 3), (26, 3), (23, 3), (20, 3), (17, 3), (14, 3), (11, 3),
           (8, 3), (5, 3), (2, 3), (0, 2)]


def _row_kernel(len_ref, xa_ref, xb_ref, W1_ref, b1_ref, W2_ref, out_ref,
                s_scr):
    b = pl.program_id(0)
    T = _RB * _CB
    l = len_ref[b]

    for c in range(_NCH):
        @pl.when(l > c * _CHUNK)
        def _():
            half = xa_ref if c < _NCH // 2 else xb_ref
            cc_ = c if c < _NCH // 2 else c - _NCH // 2
            xc = half[0][cc_ * _CHUNK:(cc_ + 1) * _CHUNK, :]
            h = jnp.tanh(
                jnp.dot(xc, W1_ref[...], preferred_element_type=jnp.float32)
                + b1_ref[...]
            )
            sc = jnp.dot(h, W2_ref[...], preferred_element_type=jnp.float32)
            s_scr[c * 8:(c + 1) * 8, :] = sc.reshape(8, _CB)

    r = lax.broadcasted_iota(jnp.int32, (_RB, _CB), 0)
    cc0 = lax.broadcasted_iota(jnp.int32, (_RB, _CB), 1)
    t = r * _CB + cc0
    s2 = jnp.where(t < l, s_scr[...], -jnp.inf)

    i = lax.bitcast_convert_type(s2, jnp.int32)
    skey = i ^ (lax.shift_right_arithmetic(i, 31) & jnp.int32(0x7FFFFFFF))

    lf = l.astype(jnp.float32) * jnp.float32(_FRAC)
    ki = lf.astype(jnp.int32)
    ki = ki + (ki.astype(jnp.float32) < lf).astype(jnp.int32)
    k = jnp.clip(ki, _MIN_K, T)

    p = jnp.int32(0)
    for shift, width in _ROUNDS:
        n = (1 << width) - 1
        oks = []
        for j in range(1, n + 1):
            cpat = p | (jnp.int32(j) << shift)
            cval = cpat ^ jnp.int32(_INT_MIN)
            cnt = jnp.sum((skey >= cval).astype(jnp.int32))
            oks.append((cnt >= k).astype(jnp.int32))
        j_star = oks[0]
        for o in oks[1:]:
            j_star = j_star + o
        p = p | (j_star << shift)
    theta = p ^ jnp.int32(_INT_MIN)

    cgt = jnp.sum((skey > theta).astype(jnp.int32))
    m = (k - cgt).astype(jnp.float32)

    eq = (skey == theta).astype(jnp.float32)
    cc = lax.broadcasted_iota(jnp.int32, (_CB, _CB), 0)
    cr = lax.broadcasted_iota(jnp.int32, (_CB, _CB), 1)
    lt_incl = (cc <= cr).astype(jnp.float32)
    lane_incl = jnp.dot(eq, lt_incl, preferred_element_type=jnp.float32)
    row_tot = jnp.sum(eq, axis=1, keepdims=True)
    ar = lax.broadcasted_iota(jnp.int32, (_RB, _RB), 0)
    ac = lax.broadcasted_iota(jnp.int32, (_RB, _RB), 1)
    strict = (ac < ar).astype(jnp.float32)
    row_excl = jnp.dot(strict, row_tot, preferred_element_type=jnp.float32)
    rank_excl = row_excl + lane_incl - eq

    w = jnp.where(
        (skey > theta) | ((skey == theta) & (rank_excl < m)),
        jnp.float32(1.0),
        jnp.float32(0.0),
    )

    wrow = w.reshape(1, T)
    pooled = (
        jnp.dot(wrow[:, : T // 2], xa_ref[0],
                preferred_element_type=jnp.float32)
        + jnp.dot(wrow[:, T // 2:], xb_ref[0],
                  preferred_element_type=jnp.float32)
    )
    out_ref[0] = pooled / k.astype(jnp.float32)


def kernel(x, lengths, W1, b1, W2):
    B, T, D = x.shape
    lengths = lengths.astype(jnp.int32)
    b1r = b1.reshape(1, _HIDDEN).astype(jnp.float32)
    xh = x.reshape(2 * B, T // 2, D)
    return pl.pallas_call(
        _row_kernel,
        grid=(B,),
        in_specs=[
            pl.BlockSpec(memory_space=pltpu.SMEM),
            pl.BlockSpec((1, T // 2, D), lambda b: (2 * b, 0, 0)),
            pl.BlockSpec((1, T // 2, D), lambda b: (2 * b + 1, 0, 0)),
            pl.BlockSpec((D, _HIDDEN), lambda b: (0, 0)),
            pl.BlockSpec((1, _HIDDEN), lambda b: (0, 0)),
            pl.BlockSpec((_HIDDEN, 1), lambda b: (0, 0)),
        ],
        out_specs=pl.BlockSpec((1, 1, D), lambda b: (b, 0, 0)),
        out_shape=jax.ShapeDtypeStruct((B, 1, D), jnp.float32),
        scratch_shapes=[pltpu.VMEM((_RB, _CB), jnp.float32)],
    )(lengths, xh, xh, W1, b1r, W2).reshape(B, D)
